# Initial kernel scaffold; baseline (speedup 1.0000x reference)
import jax
import jax.numpy as jnp
from jax.experimental import pallas as pl


def kernel(x, edge_index, W1, b1, W2, b2, W3, b3):
    raise NotImplementedError("write your pallas kernel here")


# speedup vs baseline: 10.0609x; 10.0609x over previous
import functools

import jax
import jax.numpy as jnp
from jax import lax
from jax.experimental import pallas as pl
from jax.experimental.pallas import tpu as pltpu
from jax.experimental.pallas import tpu_sc as plsc

_N = 10000
_E = 320000
_D_IN = 128
_D_H = 256
_D_OUT = 128

_NSC = 2
_NTILE = 16
_CH = 125
_RPT = 624
_TAIL0 = _RPT * _NTILE
_TAILN = _N - _TAIL0
_RB = 1000
_NBLK = _N // _RB


def _sc_mesh():
    return plsc.VectorSubcoreMesh(core_axis_name="c", subcore_axis_name="s")


def _rowsplit_copy(s, fn):
    fn(pl.multiple_of(s * _RPT, _RPT), _RPT)

    @pl.when(s == _NTILE - 1)
    def _():
        fn(_TAIL0, _TAILN)


def _sc_degree(dst_t, zeros_nk, ones_ck):
    nch = dst_t.shape[1]

    @functools.partial(
        pl.kernel,
        out_type=jax.ShapeDtypeStruct((_NSC, _N, 16), jnp.float32),
        mesh=_sc_mesh(),
        compiler_params=pltpu.CompilerParams(use_tc_tiling_on_sc=False),
        scratch_types=[
            pltpu.VMEM((nch, _CH), jnp.int32),
            pltpu.VMEM((_CH, 16), jnp.float32),
            pltpu.VMEM_SHARED((_N, 16), jnp.float32),
            pltpu.SemaphoreType.DMA,
        ],
    )
    def k(dst_hbm, zeros_hbm, ones_hbm, out_hbm, idx_v, ones_v, acc, sem):
        c = lax.axis_index("c")
        s = lax.axis_index("s")
        wid = c * _NTILE + s
        _rowsplit_copy(s, lambda o, n: pltpu.sync_copy(
            zeros_hbm.at[pl.ds(o, n)], acc.at[pl.ds(o, n)]))
        pltpu.sync_copy(dst_hbm.at[wid], idx_v)
        pltpu.sync_copy(ones_hbm, ones_v)
        plsc.subcore_barrier()

        def body(j, carry):
            pltpu.sync_copy(ones_v, acc.at[idx_v.at[j]], add=True)
            return carry

        lax.fori_loop(0, nch, body, 0)
        plsc.subcore_barrier()

        @pl.when(c == 0)
        def _():
            _rowsplit_copy(s, lambda o, n: pltpu.sync_copy(
                acc.at[pl.ds(o, n)], out_hbm.at[0].at[pl.ds(o, n)]))

        @pl.when(c == 1)
        def _():
            _rowsplit_copy(s, lambda o, n: pltpu.sync_copy(
                acc.at[pl.ds(o, n)], out_hbm.at[1].at[pl.ds(o, n)]))

    return k(dst_t, zeros_nk, ones_ck)


def _sc_aggregate(zp, src_t, dst_t):
    dq = zp.shape[2]
    nch = src_t.shape[1]

    @functools.partial(
        pl.kernel,
        out_type=jax.ShapeDtypeStruct((4, _N, dq), jnp.float32),
        mesh=_sc_mesh(),
        compiler_params=pltpu.CompilerParams(use_tc_tiling_on_sc=False),
        scratch_types=[
            pltpu.VMEM((nch, _CH), jnp.int32),
            pltpu.VMEM((nch, _CH), jnp.int32),
            pltpu.VMEM((_CH, dq), jnp.float32),
            pltpu.VMEM_SHARED((_N, dq), jnp.float32),
            pltpu.SemaphoreType.DMA,
        ],
    )
    def k(zp_hbm, src_hbm, dst_hbm, out_hbm, src_v, dst_v, gbuf, acc, sem):
        c = lax.axis_index("c")
        s = lax.axis_index("s")
        pltpu.sync_copy(src_hbm.at[s], src_v)
        pltpu.sync_copy(dst_hbm.at[s], dst_v)

        def run_plane(q):
            z_hbm = zp_hbm.at[q]
            plsc.subcore_barrier()
            _rowsplit_copy(s, lambda o, n: pltpu.sync_copy(
                z_hbm.at[pl.ds(o, n)], acc.at[pl.ds(o, n)]))
            plsc.subcore_barrier()

            def body(j, carry):
                pltpu.async_copy(z_hbm.at[src_v.at[j]], gbuf, sem).wait()
                pltpu.sync_copy(gbuf, acc.at[dst_v.at[j]], add=True)
                return carry

            lax.fori_loop(0, nch, body, 0)
            plsc.subcore_barrier()
            _rowsplit_copy(s, lambda o, n: pltpu.sync_copy(
                acc.at[pl.ds(o, n)], out_hbm.at[q].at[pl.ds(o, n)]))

        @pl.when(c == 0)
        def _():
            run_plane(0)
            run_plane(1)

        @pl.when(c == 1)
        def _():
            run_plane(2)
            run_plane(3)

    return k(zp, src_t, dst_t)


def _dinv_from(deg_blk):
    deg = deg_blk[0, :, 0:1] + deg_blk[1, :, 0:1] + 1.0
    return lax.rsqrt(deg)


def _split_planes(out_ref, z, d_out):
    dq = d_out // 4
    for q in range(4):
        out_ref[q, :, :] = z[:, q * dq:(q + 1) * dq]


def _tc_layer1(x, w1, degp):
    def body(x_ref, deg_ref, w_ref, out_ref):
        dinv = _dinv_from(deg_ref[...])
        z = dinv * jnp.dot(x_ref[...], w_ref[...], preferred_element_type=jnp.float32)
        _split_planes(out_ref, z, _D_H)

    return pl.pallas_call(
        body,
        grid=(_NBLK,),
        in_specs=[
            pl.BlockSpec((_RB, _D_IN), lambda i: (i, 0)),
            pl.BlockSpec((_NSC, _RB, 16), lambda i: (0, i, 0)),
            pl.BlockSpec((_D_IN, _D_H), lambda i: (0, 0)),
        ],
        out_specs=pl.BlockSpec((4, _RB, _D_H // 4), lambda i: (0, i, 0)),
        out_shape=jax.ShapeDtypeStruct((4, _N, _D_H // 4), jnp.float32),
    )(x, degp, w1)


def _tc_mid(up, degp, b, w):
    d_in = up.shape[2] * 4
    d_out = w.shape[1]

    def body(u_ref, deg_ref, b_ref, w_ref, out_ref):
        dinv = _dinv_from(deg_ref[...])
        u = u_ref[...]
        uc = jnp.concatenate([u[0], u[1], u[2], u[3]], axis=1)
        h = jnp.maximum(dinv * uc + b_ref[...], 0.0)
        z = dinv * jnp.dot(h, w_ref[...], preferred_element_type=jnp.float32)
        _split_planes(out_ref, z, d_out)

    return pl.pallas_call(
        body,
        grid=(_NBLK,),
        in_specs=[
            pl.BlockSpec((4, _RB, d_in // 4), lambda i: (0, i, 0)),
            pl.BlockSpec((_NSC, _RB, 16), lambda i: (0, i, 0)),
            pl.BlockSpec((1, d_in), lambda i: (0, 0)),
            pl.BlockSpec((d_in, d_out), lambda i: (0, 0)),
        ],
        out_specs=pl.BlockSpec((4, _RB, d_out // 4), lambda i: (0, i, 0)),
        out_shape=jax.ShapeDtypeStruct((4, _N, d_out // 4), jnp.float32),
    )(up, degp, b.reshape(1, -1), w)


def _tc_final(up, degp, b):
    d_out = up.shape[2] * 4

    def body(u_ref, deg_ref, b_ref, out_ref):
        dinv = _dinv_from(deg_ref[...])
        u = u_ref[...]
        uc = jnp.concatenate([u[0], u[1], u[2], u[3]], axis=1)
        out_ref[...] = dinv * uc + b_ref[...]

    return pl.pallas_call(
        body,
        grid=(_NBLK,),
        in_specs=[
            pl.BlockSpec((4, _RB, d_out // 4), lambda i: (0, i, 0)),
            pl.BlockSpec((_NSC, _RB, 16), lambda i: (0, i, 0)),
            pl.BlockSpec((1, d_out), lambda i: (0, 0)),
        ],
        out_specs=pl.BlockSpec((_RB, d_out), lambda i: (i, 0)),
        out_shape=jax.ShapeDtypeStruct((_N, d_out), jnp.float32),
    )(up, degp, b.reshape(1, -1))


def kernel(x, edge_index, W1, b1, W2, b2, W3, b3):
    src = edge_index[0]
    dst = edge_index[1]
    nch_deg = _E // (_NSC * _NTILE * _CH)
    nch_agg = _E // (_NTILE * _CH)
    dst_deg = dst.reshape(_NSC * _NTILE, nch_deg, _CH)
    src_agg = src.reshape(_NTILE, nch_agg, _CH)
    dst_agg = dst.reshape(_NTILE, nch_agg, _CH)
    zeros_nk = jnp.zeros((_N, 16), jnp.float32)
    ones_ck = jnp.ones((_CH, 16), jnp.float32)

    degp = _sc_degree(dst_deg, zeros_nk, ones_ck)
    z1 = _tc_layer1(x, W1, degp)
    u1 = _sc_aggregate(z1, src_agg, dst_agg)
    z2 = _tc_mid(u1, degp, b1, W2)
    u2 = _sc_aggregate(z2, src_agg, dst_agg)
    z3 = _tc_mid(u2, degp, b2, W3)
    u3 = _sc_aggregate(z3, src_agg, dst_agg)
    return _tc_final(u3, degp, b3)


# speedup vs baseline: 16.1923x; 1.6094x over previous
import functools

import jax
import jax.numpy as jnp
from jax import lax
from jax.experimental import pallas as pl
from jax.experimental.pallas import tpu as pltpu
from jax.experimental.pallas import tpu_sc as plsc

_N = 10000
_E = 320000
_D_IN = 128
_D_H = 256
_D_OUT = 128

_NSC = 2
_NTILE = 16
_CH = 125
_RPT = 624
_TAIL0 = _RPT * _NTILE
_TAILN = _N - _TAIL0
_RB = 1000
_NBLK = _N // _RB


def _sc_mesh():
    return plsc.VectorSubcoreMesh(core_axis_name="c", subcore_axis_name="s")


def _rowsplit_copy(s, fn):
    fn(pl.multiple_of(s * _RPT, _RPT), _RPT)

    @pl.when(s == _NTILE - 1)
    def _():
        fn(_TAIL0, _TAILN)


def _sc_degree(dst_t, zeros_nk, ones_ck):
    nch = dst_t.shape[1]

    @functools.partial(
        pl.kernel,
        out_type=jax.ShapeDtypeStruct((_NSC, _N, 16), jnp.float32),
        mesh=_sc_mesh(),
        compiler_params=pltpu.CompilerParams(use_tc_tiling_on_sc=False),
        scratch_types=[
            pltpu.VMEM((nch, _CH), jnp.int32),
            pltpu.VMEM((_CH, 16), jnp.float32),
            pltpu.VMEM_SHARED((_N, 16), jnp.float32),
            pltpu.SemaphoreType.DMA,
        ],
    )
    def k(dst_hbm, zeros_hbm, ones_hbm, out_hbm, idx_v, ones_v, acc, sem):
        c = lax.axis_index("c")
        s = lax.axis_index("s")
        wid = c * _NTILE + s
        _rowsplit_copy(s, lambda o, n: pltpu.sync_copy(
            zeros_hbm.at[pl.ds(o, n)], acc.at[pl.ds(o, n)]))
        pltpu.sync_copy(dst_hbm.at[wid], idx_v)
        pltpu.sync_copy(ones_hbm, ones_v)
        plsc.subcore_barrier()

        def body(j, carry):
            pltpu.sync_copy(ones_v, acc.at[idx_v.at[j]], add=True)
            return carry

        lax.fori_loop(0, nch, body, 0)
        plsc.subcore_barrier()

        @pl.when(c == 0)
        def _():
            _rowsplit_copy(s, lambda o, n: pltpu.sync_copy(
                acc.at[pl.ds(o, n)], out_hbm.at[0].at[pl.ds(o, n)]))

        @pl.when(c == 1)
        def _():
            _rowsplit_copy(s, lambda o, n: pltpu.sync_copy(
                acc.at[pl.ds(o, n)], out_hbm.at[1].at[pl.ds(o, n)]))

    return k(dst_t, zeros_nk, ones_ck)


def _sc_aggregate(zp, src_t, dst_t):
    dq = zp.shape[2]
    nch = src_t.shape[1]

    @functools.partial(
        pl.kernel,
        out_type=jax.ShapeDtypeStruct((4, _N, dq), jnp.float32),
        mesh=_sc_mesh(),
        compiler_params=pltpu.CompilerParams(use_tc_tiling_on_sc=False),
        scratch_types=[
            pltpu.VMEM((nch, _CH), jnp.int32),
            pltpu.VMEM((nch, _CH), jnp.int32),
            pltpu.VMEM((_CH, dq), jnp.float32),
            pltpu.VMEM((_CH, dq), jnp.float32),
            pltpu.VMEM_SHARED((_N, dq), jnp.float32),
            pltpu.SemaphoreType.DMA,
            pltpu.SemaphoreType.DMA,
        ],
    )
    def k(zp_hbm, src_hbm, dst_hbm, out_hbm, src_v, dst_v, gbuf0, gbuf1, acc,
          gsem0, gsem1):
        c = lax.axis_index("c")
        s = lax.axis_index("s")
        pltpu.sync_copy(src_hbm.at[s], src_v)
        pltpu.sync_copy(dst_hbm.at[s], dst_v)
        npairs = nch // 2

        def run_plane(q):
            z_hbm = zp_hbm.at[q]
            plsc.subcore_barrier()
            _rowsplit_copy(s, lambda o, n: pltpu.sync_copy(
                z_hbm.at[pl.ds(o, n)], acc.at[pl.ds(o, n)]))
            plsc.subcore_barrier()

            pltpu.async_copy(z_hbm.at[src_v.at[0]], gbuf0, gsem0)

            def body(jj, carry):
                a = jj * 2
                b = a + 1
                pltpu.async_copy(z_hbm.at[src_v.at[b]], gbuf1, gsem1)
                pltpu.make_async_copy(z_hbm.at[src_v.at[a]], gbuf0, gsem0).wait()
                pltpu.sync_copy(gbuf0, acc.at[dst_v.at[a]], add=True)

                @pl.when(jj + 1 < npairs)
                def _():
                    pltpu.async_copy(z_hbm.at[src_v.at[a + 2]], gbuf0, gsem0)

                pltpu.make_async_copy(z_hbm.at[src_v.at[b]], gbuf1, gsem1).wait()
                pltpu.sync_copy(gbuf1, acc.at[dst_v.at[b]], add=True)
                return carry

            lax.fori_loop(0, npairs, body, 0)
            plsc.subcore_barrier()
            _rowsplit_copy(s, lambda o, n: pltpu.sync_copy(
                acc.at[pl.ds(o, n)], out_hbm.at[q].at[pl.ds(o, n)]))

        @pl.when(c == 0)
        def _():
            run_plane(0)
            run_plane(1)

        @pl.when(c == 1)
        def _():
            run_plane(2)
            run_plane(3)

    return k(zp, src_t, dst_t)


def _dinv_from(deg_blk):
    deg = deg_blk[0, :, 0:1] + deg_blk[1, :, 0:1] + 1.0
    return lax.rsqrt(deg)


def _split_planes(out_ref, z, d_out):
    dq = d_out // 4
    for q in range(4):
        out_ref[q, :, :] = z[:, q * dq:(q + 1) * dq]


def _tc_layer1(x, w1, degp):
    def body(x_ref, deg_ref, w_ref, out_ref):
        dinv = _dinv_from(deg_ref[...])
        z = dinv * jnp.dot(x_ref[...], w_ref[...], preferred_element_type=jnp.float32)
        _split_planes(out_ref, z, _D_H)

    return pl.pallas_call(
        body,
        grid=(_NBLK,),
        in_specs=[
            pl.BlockSpec((_RB, _D_IN), lambda i: (i, 0)),
            pl.BlockSpec((_NSC, _RB, 16), lambda i: (0, i, 0)),
            pl.BlockSpec((_D_IN, _D_H), lambda i: (0, 0)),
        ],
        out_specs=pl.BlockSpec((4, _RB, _D_H // 4), lambda i: (0, i, 0)),
        out_shape=jax.ShapeDtypeStruct((4, _N, _D_H // 4), jnp.float32),
    )(x, degp, w1)


def _tc_mid(up, degp, b, w):
    d_in = up.shape[2] * 4
    d_out = w.shape[1]

    def body(u_ref, deg_ref, b_ref, w_ref, out_ref):
        dinv = _dinv_from(deg_ref[...])
        u = u_ref[...]
        uc = jnp.concatenate([u[0], u[1], u[2], u[3]], axis=1)
        h = jnp.maximum(dinv * uc + b_ref[...], 0.0)
        z = dinv * jnp.dot(h, w_ref[...], preferred_element_type=jnp.float32)
        _split_planes(out_ref, z, d_out)

    return pl.pallas_call(
        body,
        grid=(_NBLK,),
        in_specs=[
            pl.BlockSpec((4, _RB, d_in // 4), lambda i: (0, i, 0)),
            pl.BlockSpec((_NSC, _RB, 16), lambda i: (0, i, 0)),
            pl.BlockSpec((1, d_in), lambda i: (0, 0)),
            pl.BlockSpec((d_in, d_out), lambda i: (0, 0)),
        ],
        out_specs=pl.BlockSpec((4, _RB, d_out // 4), lambda i: (0, i, 0)),
        out_shape=jax.ShapeDtypeStruct((4, _N, d_out // 4), jnp.float32),
    )(up, degp, b.reshape(1, -1), w)


def _tc_final(up, degp, b):
    d_out = up.shape[2] * 4

    def body(u_ref, deg_ref, b_ref, out_ref):
        dinv = _dinv_from(deg_ref[...])
        u = u_ref[...]
        uc = jnp.concatenate([u[0], u[1], u[2], u[3]], axis=1)
        out_ref[...] = dinv * uc + b_ref[...]

    return pl.pallas_call(
        body,
        grid=(_NBLK,),
        in_specs=[
            pl.BlockSpec((4, _RB, d_out // 4), lambda i: (0, i, 0)),
            pl.BlockSpec((_NSC, _RB, 16), lambda i: (0, i, 0)),
            pl.BlockSpec((1, d_out), lambda i: (0, 0)),
        ],
        out_specs=pl.BlockSpec((_RB, d_out), lambda i: (i, 0)),
        out_shape=jax.ShapeDtypeStruct((_N, d_out), jnp.float32),
    )(up, degp, b.reshape(1, -1))


def kernel(x, edge_index, W1, b1, W2, b2, W3, b3):
    src = edge_index[0]
    dst = edge_index[1]
    nch_deg = _E // (_NSC * _NTILE * _CH)
    nch_agg = _E // (_NTILE * _CH)
    dst_deg = dst.reshape(_NSC * _NTILE, nch_deg, _CH)
    src_agg = src.reshape(_NTILE, nch_agg, _CH)
    dst_agg = dst.reshape(_NTILE, nch_agg, _CH)
    zeros_nk = jnp.zeros((_N, 16), jnp.float32)
    ones_ck = jnp.ones((_CH, 16), jnp.float32)

    degp = _sc_degree(dst_deg, zeros_nk, ones_ck)
    z1 = _tc_layer1(x, W1, degp)
    u1 = _sc_aggregate(z1, src_agg, dst_agg)
    z2 = _tc_mid(u1, degp, b1, W2)
    u2 = _sc_aggregate(z2, src_agg, dst_agg)
    z3 = _tc_mid(u2, degp, b2, W3)
    u3 = _sc_aggregate(z3, src_agg, dst_agg)
    return _tc_final(u3, degp, b3)


# speedup vs baseline: 18.4972x; 1.1423x over previous
import functools

import jax
import jax.numpy as jnp
from jax import lax
from jax.experimental import pallas as pl
from jax.experimental.pallas import tpu as pltpu
from jax.experimental.pallas import tpu_sc as plsc

_N = 10000
_E = 320000
_D_IN = 128
_D_H = 256
_D_OUT = 128

_NSC = 2
_NTILE = 16
_CH = 125
_RPT = 624
_TAIL0 = _RPT * _NTILE
_TAILN = _N - _TAIL0
_RB = 1000
_NBLK = _N // _RB


def _sc_mesh():
    return plsc.VectorSubcoreMesh(core_axis_name="c", subcore_axis_name="s")


def _rowsplit_copy(s, fn):
    fn(pl.multiple_of(s * _RPT, _RPT), _RPT)

    @pl.when(s == _NTILE - 1)
    def _():
        fn(_TAIL0, _TAILN)


def _sc_degree(dst_t, zeros_nk, ones_ck):
    nch = dst_t.shape[1]

    @functools.partial(
        pl.kernel,
        out_type=jax.ShapeDtypeStruct((_NSC, _N, 16), jnp.float32),
        mesh=_sc_mesh(),
        compiler_params=pltpu.CompilerParams(use_tc_tiling_on_sc=False),
        scratch_types=[
            pltpu.VMEM((nch, _CH), jnp.int32),
            pltpu.VMEM((_CH, 16), jnp.float32),
            pltpu.VMEM_SHARED((_N, 16), jnp.float32),
            pltpu.SemaphoreType.DMA,
        ],
    )
    def k(dst_hbm, zeros_hbm, ones_hbm, out_hbm, idx_v, ones_v, acc, sem):
        c = lax.axis_index("c")
        s = lax.axis_index("s")
        wid = c * _NTILE + s
        _rowsplit_copy(s, lambda o, n: pltpu.sync_copy(
            zeros_hbm.at[pl.ds(o, n)], acc.at[pl.ds(o, n)]))
        pltpu.sync_copy(dst_hbm.at[wid], idx_v)
        pltpu.sync_copy(ones_hbm, ones_v)
        plsc.subcore_barrier()

        def body(j, carry):
            pltpu.sync_copy(ones_v, acc.at[idx_v.at[j]], add=True)
            return carry

        lax.fori_loop(0, nch, body, 0)
        plsc.subcore_barrier()

        @pl.when(c == 0)
        def _():
            _rowsplit_copy(s, lambda o, n: pltpu.sync_copy(
                acc.at[pl.ds(o, n)], out_hbm.at[0].at[pl.ds(o, n)]))

        @pl.when(c == 1)
        def _():
            _rowsplit_copy(s, lambda o, n: pltpu.sync_copy(
                acc.at[pl.ds(o, n)], out_hbm.at[1].at[pl.ds(o, n)]))

    return k(dst_t, zeros_nk, ones_ck)


def _sc_aggregate(zp, src_t, dst_t):
    dq = zp.shape[2]
    nch = src_t.shape[1]

    @functools.partial(
        pl.kernel,
        out_type=jax.ShapeDtypeStruct((4, _N, dq), jnp.float32),
        mesh=_sc_mesh(),
        compiler_params=pltpu.CompilerParams(use_tc_tiling_on_sc=False),
        scratch_types=[
            pltpu.VMEM((nch, _CH), jnp.int32),
            pltpu.VMEM((nch, _CH), jnp.int32),
            [pltpu.VMEM((_CH, dq), jnp.float32)] * 4,
            pltpu.VMEM_SHARED((_N, dq), jnp.float32),
            [pltpu.SemaphoreType.DMA] * 4,
            [pltpu.SemaphoreType.DMA] * 4,
        ],
    )
    def k(zp_hbm, src_hbm, dst_hbm, out_hbm, src_v, dst_v, gbufs, acc,
          gsems, ssems):
        c = lax.axis_index("c")
        s = lax.axis_index("s")
        pltpu.sync_copy(src_hbm.at[s], src_v)
        pltpu.sync_copy(dst_hbm.at[s], dst_v)
        nquads = nch // 4

        def run_plane(q):
            z_hbm = zp_hbm.at[q]
            plsc.subcore_barrier()
            _rowsplit_copy(s, lambda o, n: pltpu.sync_copy(
                z_hbm.at[pl.ds(o, n)], acc.at[pl.ds(o, n)]))
            plsc.subcore_barrier()

            for i in range(4):
                pltpu.async_copy(z_hbm.at[src_v.at[i]], gbufs[i], gsems[i])

            def body(jj, carry):
                base = jj * 4
                for i in range(4):
                    j = base + i
                    pltpu.make_async_copy(
                        z_hbm.at[src_v.at[j]], gbufs[i], gsems[i]).wait()
                    pltpu.async_copy(gbufs[i], acc.at[dst_v.at[j]], ssems[i],
                                     add=True)
                for i in range(4):
                    j = base + i
                    pltpu.make_async_copy(
                        gbufs[i], acc.at[dst_v.at[j]], ssems[i]).wait()

                    @pl.when(j + 4 < nch)
                    def _(j=j, i=i):
                        pltpu.async_copy(
                            z_hbm.at[src_v.at[j + 4]], gbufs[i], gsems[i])

                return carry

            lax.fori_loop(0, nquads, body, 0)
            plsc.subcore_barrier()
            _rowsplit_copy(s, lambda o, n: pltpu.sync_copy(
                acc.at[pl.ds(o, n)], out_hbm.at[q].at[pl.ds(o, n)]))

        @pl.when(c == 0)
        def _():
            run_plane(0)
            run_plane(1)

        @pl.when(c == 1)
        def _():
            run_plane(2)
            run_plane(3)

    return k(zp, src_t, dst_t)


def _dinv_from(deg_blk):
    deg = deg_blk[0, :, 0:1] + deg_blk[1, :, 0:1] + 1.0
    return lax.rsqrt(deg)


def _split_planes(out_ref, z, d_out):
    dq = d_out // 4
    for q in range(4):
        out_ref[q, :, :] = z[:, q * dq:(q + 1) * dq]


def _tc_layer1(x, w1, degp):
    def body(x_ref, deg_ref, w_ref, out_ref):
        dinv = _dinv_from(deg_ref[...])
        z = dinv * jnp.dot(x_ref[...], w_ref[...], preferred_element_type=jnp.float32)
        _split_planes(out_ref, z, _D_H)

    return pl.pallas_call(
        body,
        grid=(_NBLK,),
        in_specs=[
            pl.BlockSpec((_RB, _D_IN), lambda i: (i, 0)),
            pl.BlockSpec((_NSC, _RB, 16), lambda i: (0, i, 0)),
            pl.BlockSpec((_D_IN, _D_H), lambda i: (0, 0)),
        ],
        out_specs=pl.BlockSpec((4, _RB, _D_H // 4), lambda i: (0, i, 0)),
        out_shape=jax.ShapeDtypeStruct((4, _N, _D_H // 4), jnp.float32),
    )(x, degp, w1)


def _tc_mid(up, degp, b, w):
    d_in = up.shape[2] * 4
    d_out = w.shape[1]

    def body(u_ref, deg_ref, b_ref, w_ref, out_ref):
        dinv = _dinv_from(deg_ref[...])
        u = u_ref[...]
        uc = jnp.concatenate([u[0], u[1], u[2], u[3]], axis=1)
        h = jnp.maximum(dinv * uc + b_ref[...], 0.0)
        z = dinv * jnp.dot(h, w_ref[...], preferred_element_type=jnp.float32)
        _split_planes(out_ref, z, d_out)

    return pl.pallas_call(
        body,
        grid=(_NBLK,),
        in_specs=[
            pl.BlockSpec((4, _RB, d_in // 4), lambda i: (0, i, 0)),
            pl.BlockSpec((_NSC, _RB, 16), lambda i: (0, i, 0)),
            pl.BlockSpec((1, d_in), lambda i: (0, 0)),
            pl.BlockSpec((d_in, d_out), lambda i: (0, 0)),
        ],
        out_specs=pl.BlockSpec((4, _RB, d_out // 4), lambda i: (0, i, 0)),
        out_shape=jax.ShapeDtypeStruct((4, _N, d_out // 4), jnp.float32),
    )(up, degp, b.reshape(1, -1), w)


def _tc_final(up, degp, b):
    d_out = up.shape[2] * 4

    def body(u_ref, deg_ref, b_ref, out_ref):
        dinv = _dinv_from(deg_ref[...])
        u = u_ref[...]
        uc = jnp.concatenate([u[0], u[1], u[2], u[3]], axis=1)
        out_ref[...] = dinv * uc + b_ref[...]

    return pl.pallas_call(
        body,
        grid=(_NBLK,),
        in_specs=[
            pl.BlockSpec((4, _RB, d_out // 4), lambda i: (0, i, 0)),
            pl.BlockSpec((_NSC, _RB, 16), lambda i: (0, i, 0)),
            pl.BlockSpec((1, d_out), lambda i: (0, 0)),
        ],
        out_specs=pl.BlockSpec((_RB, d_out), lambda i: (i, 0)),
        out_shape=jax.ShapeDtypeStruct((_N, d_out), jnp.float32),
    )(up, degp, b.reshape(1, -1))


def kernel(x, edge_index, W1, b1, W2, b2, W3, b3):
    src = edge_index[0]
    dst = edge_index[1]
    nch_deg = _E // (_NSC * _NTILE * _CH)
    nch_agg = _E // (_NTILE * _CH)
    dst_deg = dst.reshape(_NSC * _NTILE, nch_deg, _CH)
    src_agg = src.reshape(_NTILE, nch_agg, _CH)
    dst_agg = dst.reshape(_NTILE, nch_agg, _CH)
    zeros_nk = jnp.zeros((_N, 16), jnp.float32)
    ones_ck = jnp.ones((_CH, 16), jnp.float32)

    degp = _sc_degree(dst_deg, zeros_nk, ones_ck)
    z1 = _tc_layer1(x, W1, degp)
    u1 = _sc_aggregate(z1, src_agg, dst_agg)
    z2 = _tc_mid(u1, degp, b1, W2)
    u2 = _sc_aggregate(z2, src_agg, dst_agg)
    z3 = _tc_mid(u2, degp, b2, W3)
    u3 = _sc_aggregate(z3, src_agg, dst_agg)
    return _tc_final(u3, degp, b3)


# speedup vs baseline: 19.7705x; 1.0688x over previous
import functools

import jax
import jax.numpy as jnp
from jax import lax
from jax.experimental import pallas as pl
from jax.experimental.pallas import tpu as pltpu
from jax.experimental.pallas import tpu_sc as plsc

_N = 10000
_E = 320000
_D_IN = 128
_D_H = 256
_D_OUT = 128

_NSC = 2
_NTILE = 16
_CH = 125
_RPT = 624
_TAIL0 = _RPT * _NTILE
_TAILN = _N - _TAIL0
_RB = 1000
_NBLK = _N // _RB


def _sc_mesh():
    return plsc.VectorSubcoreMesh(core_axis_name="c", subcore_axis_name="s")


def _rowsplit_copy(s, fn):
    fn(pl.multiple_of(s * _RPT, _RPT), _RPT)

    @pl.when(s == _NTILE - 1)
    def _():
        fn(_TAIL0, _TAILN)


def _sc_degree(dst_t, zeros_nk, ones_ck):
    nch = dst_t.shape[1]

    @functools.partial(
        pl.kernel,
        out_type=jax.ShapeDtypeStruct((_NSC, _N, 16), jnp.float32),
        mesh=_sc_mesh(),
        compiler_params=pltpu.CompilerParams(use_tc_tiling_on_sc=False),
        scratch_types=[
            pltpu.VMEM((nch, _CH), jnp.int32),
            pltpu.VMEM((_CH, 16), jnp.float32),
            pltpu.VMEM_SHARED((_N, 16), jnp.float32),
            pltpu.SemaphoreType.DMA,
        ],
    )
    def k(dst_hbm, zeros_hbm, ones_hbm, out_hbm, idx_v, ones_v, acc, sem):
        c = lax.axis_index("c")
        s = lax.axis_index("s")
        wid = c * _NTILE + s
        _rowsplit_copy(s, lambda o, n: pltpu.sync_copy(
            zeros_hbm.at[pl.ds(o, n)], acc.at[pl.ds(o, n)]))
        pltpu.sync_copy(dst_hbm.at[wid], idx_v)
        pltpu.sync_copy(ones_hbm, ones_v)
        plsc.subcore_barrier()

        def body(j, carry):
            pltpu.sync_copy(ones_v, acc.at[idx_v.at[j]], add=True)
            return carry

        lax.fori_loop(0, nch, body, 0)
        plsc.subcore_barrier()

        @pl.when(c == 0)
        def _():
            _rowsplit_copy(s, lambda o, n: pltpu.sync_copy(
                acc.at[pl.ds(o, n)], out_hbm.at[0].at[pl.ds(o, n)]))

        @pl.when(c == 1)
        def _():
            _rowsplit_copy(s, lambda o, n: pltpu.sync_copy(
                acc.at[pl.ds(o, n)], out_hbm.at[1].at[pl.ds(o, n)]))

    return k(dst_t, zeros_nk, ones_ck)


def _edge_pipeline(table, src_slab, dst_slab, acc, sblk, dblk, gbuf,
                   isem, gsem, ssem, nch):
    nsb = nch // 16

    def idx_load(bi, p):
        off = pl.multiple_of(bi * 8, 8)
        pltpu.async_copy(src_slab.at[pl.ds(off, 8)], sblk[p], isem[p])
        pltpu.async_copy(dst_slab.at[pl.ds(off, 8)], dblk[p], isem[p])

    def idx_wait(bi, p):
        off = pl.multiple_of(bi * 8, 8)
        pltpu.make_async_copy(src_slab.at[pl.ds(off, 8)], sblk[p], isem[p]).wait()
        pltpu.make_async_copy(dst_slab.at[pl.ds(off, 8)], dblk[p], isem[p]).wait()

    def g_issue(p, r, gp):
        pltpu.async_copy(table.at[sblk[p].at[r]], gbuf[gp], gsem[gp])

    def g_wait(p, r, gp):
        pltpu.make_async_copy(table.at[sblk[p].at[r]], gbuf[gp], gsem[gp]).wait()

    def s_issue(p, r, gp):
        pltpu.async_copy(gbuf[gp], acc.at[dblk[p].at[r]], ssem[gp], add=True)

    def s_wait(p, r, gp):
        pltpu.make_async_copy(gbuf[gp], acc.at[dblk[p].at[r]], ssem[gp]).wait()

    idx_load(0, 0)
    idx_wait(0, 0)
    g_issue(0, 0, 0)
    idx_load(1, 1)

    def body(k, carry):
        for m in range(16):
            gp = m % 2
            p = m // 8
            r = m % 8
            g_wait(p, r, gp)
            s_issue(p, r, gp)
            if m == 0:
                @pl.when(k > 0)
                def _(k=k):
                    s_wait(1, 7, 1)
                    idx_load(2 * k + 1, 1)
            else:
                pm = m - 1
                s_wait(pm // 8, pm % 8, 1 - gp)
            if m == 7:
                idx_wait(2 * k + 1, 1)
            if m == 8:
                @pl.when(k < nsb - 1)
                def _(k=k):
                    idx_load(2 * k + 2, 0)
            if m < 15:
                nm = m + 1
                g_issue(nm // 8, nm % 8, nm % 2)
            else:
                @pl.when(k < nsb - 1)
                def _(k=k):
                    idx_wait(2 * k + 2, 0)
                    g_issue(0, 0, 0)
        return carry

    lax.fori_loop(0, nsb, body, 0)
    s_wait(1, 7, 1)


_AGG_SCRATCH = [
    [pltpu.VMEM((8, _CH), jnp.int32)] * 2,
    [pltpu.VMEM((8, _CH), jnp.int32)] * 2,
    [pltpu.VMEM((_CH, 128), jnp.float32)] * 2,
    pltpu.VMEM_SHARED((_N, 128), jnp.float32),
    [pltpu.SemaphoreType.DMA] * 2,
    [pltpu.SemaphoreType.DMA] * 2,
    [pltpu.SemaphoreType.DMA] * 2,
]


def _sc_agg_feature(zcat, src_off, dst_t):
    nch = dst_t.shape[1]

    @functools.partial(
        pl.kernel,
        out_type=jax.ShapeDtypeStruct((2 * _N, 128), jnp.float32),
        mesh=_sc_mesh(),
        scratch_types=_AGG_SCRATCH,
    )
    def k(z_hbm, src_hbm, dst_hbm, out_hbm, sblk, dblk, gbuf, acc,
          isem, gsem, ssem):
        c = lax.axis_index("c")
        s = lax.axis_index("s")
        wid = c * _NTILE + s
        base = pl.multiple_of(c * _N, 8)
        _rowsplit_copy(s, lambda o, n: pltpu.sync_copy(
            z_hbm.at[pl.ds(pl.multiple_of(base + o, 8), n)], acc.at[pl.ds(o, n)]))
        plsc.subcore_barrier()
        _edge_pipeline(z_hbm, src_hbm.at[wid], dst_hbm.at[s], acc,
                       sblk, dblk, gbuf, isem, gsem, ssem, nch)
        plsc.subcore_barrier()
        _rowsplit_copy(s, lambda o, n: pltpu.sync_copy(
            acc.at[pl.ds(o, n)], out_hbm.at[pl.ds(pl.multiple_of(base + o, 8), n)]))

    return k(zcat, src_off, dst_t)


def _sc_agg_edge(z3, zeros_n, src_t, dst_t):
    nch = dst_t.shape[1]

    @functools.partial(
        pl.kernel,
        out_type=jax.ShapeDtypeStruct((2 * _N, 128), jnp.float32),
        mesh=_sc_mesh(),
        scratch_types=_AGG_SCRATCH,
    )
    def k(z_hbm, zeros_hbm, src_hbm, dst_hbm, out_hbm, sblk, dblk, gbuf, acc,
          isem, gsem, ssem):
        c = lax.axis_index("c")
        s = lax.axis_index("s")
        wid = c * _NTILE + s
        base = pl.multiple_of(c * _N, 8)

        @pl.when(c == 0)
        def _():
            _rowsplit_copy(s, lambda o, n: pltpu.sync_copy(
                z_hbm.at[pl.ds(o, n)], acc.at[pl.ds(o, n)]))

        @pl.when(c == 1)
        def _():
            _rowsplit_copy(s, lambda o, n: pltpu.sync_copy(
                zeros_hbm.at[pl.ds(o, n)], acc.at[pl.ds(o, n)]))

        plsc.subcore_barrier()
        _edge_pipeline(z_hbm, src_hbm.at[wid], dst_hbm.at[wid], acc,
                       sblk, dblk, gbuf, isem, gsem, ssem, nch)
        plsc.subcore_barrier()
        _rowsplit_copy(s, lambda o, n: pltpu.sync_copy(
            acc.at[pl.ds(o, n)], out_hbm.at[pl.ds(pl.multiple_of(base + o, 8), n)]))

    return k(z3, zeros_n, src_t, dst_t)


def _dinv_from(deg_blk):
    deg = deg_blk[0, :, 0:1] + deg_blk[1, :, 0:1] + 1.0
    return lax.rsqrt(deg)


def _tc_layer1(x, w1, degp):
    def body(x_ref, deg_ref, w_ref, out_ref):
        dinv = _dinv_from(deg_ref[...])
        z = dinv * jnp.dot(x_ref[...], w_ref[...], preferred_element_type=jnp.float32)
        out_ref[0, :, :] = z[:, : _D_H // 2]
        out_ref[1, :, :] = z[:, _D_H // 2:]

    return pl.pallas_call(
        body,
        grid=(_NBLK,),
        in_specs=[
            pl.BlockSpec((_RB, _D_IN), lambda i: (i, 0)),
            pl.BlockSpec((_NSC, _RB, 16), lambda i: (0, i, 0)),
            pl.BlockSpec((_D_IN, _D_H), lambda i: (0, 0)),
        ],
        out_specs=pl.BlockSpec((2, _RB, _D_H // 2), lambda i: (0, i, 0)),
        out_shape=jax.ShapeDtypeStruct((2, _N, _D_H // 2), jnp.float32),
    )(x, degp, w1)


def _tc_mid(up, degp, b, w, split):
    d_in = up.shape[2] * 2
    d_out = w.shape[1]

    def body(u_ref, deg_ref, b_ref, w_ref, out_ref):
        dinv = _dinv_from(deg_ref[...])
        u = u_ref[...]
        uc = jnp.concatenate([u[0], u[1]], axis=1)
        h = jnp.maximum(dinv * uc + b_ref[...], 0.0)
        z = dinv * jnp.dot(h, w_ref[...], preferred_element_type=jnp.float32)
        if split:
            out_ref[0, :, :] = z[:, : d_out // 2]
            out_ref[1, :, :] = z[:, d_out // 2:]
        else:
            out_ref[...] = z

    if split:
        out_specs = pl.BlockSpec((2, _RB, d_out // 2), lambda i: (0, i, 0))
        out_shape = jax.ShapeDtypeStruct((2, _N, d_out // 2), jnp.float32)
    else:
        out_specs = pl.BlockSpec((_RB, d_out), lambda i: (i, 0))
        out_shape = jax.ShapeDtypeStruct((_N, d_out), jnp.float32)

    return pl.pallas_call(
        body,
        grid=(_NBLK,),
        in_specs=[
            pl.BlockSpec((2, _RB, d_in // 2), lambda i: (0, i, 0)),
            pl.BlockSpec((_NSC, _RB, 16), lambda i: (0, i, 0)),
            pl.BlockSpec((1, d_in), lambda i: (0, 0)),
            pl.BlockSpec((d_in, d_out), lambda i: (0, 0)),
        ],
        out_specs=out_specs,
        out_shape=out_shape,
    )(up, degp, b.reshape(1, -1), w)


def _tc_final(up, degp, b):
    d_out = up.shape[2]

    def body(u_ref, deg_ref, b_ref, out_ref):
        dinv = _dinv_from(deg_ref[...])
        u = u_ref[...]
        out_ref[...] = dinv * (u[0] + u[1]) + b_ref[...]

    return pl.pallas_call(
        body,
        grid=(_NBLK,),
        in_specs=[
            pl.BlockSpec((2, _RB, d_out), lambda i: (0, i, 0)),
            pl.BlockSpec((_NSC, _RB, 16), lambda i: (0, i, 0)),
            pl.BlockSpec((1, d_out), lambda i: (0, 0)),
        ],
        out_specs=pl.BlockSpec((_RB, d_out), lambda i: (i, 0)),
        out_shape=jax.ShapeDtypeStruct((_N, d_out), jnp.float32),
    )(up, degp, b.reshape(1, -1))


def kernel(x, edge_index, W1, b1, W2, b2, W3, b3):
    src = edge_index[0]
    dst = edge_index[1]
    nch_deg = _E // (_NSC * _NTILE * _CH)
    nch_agg = _E // (_NTILE * _CH)
    dst_deg = dst.reshape(_NSC * _NTILE, nch_deg, _CH)
    src_agg = src.reshape(_NTILE, nch_agg, _CH)
    dst_agg = dst.reshape(_NTILE, nch_agg, _CH)
    src_off = jnp.concatenate([src_agg, src_agg + _N], axis=0)
    src_e = src.reshape(_NSC * _NTILE, nch_deg, _CH)
    dst_e = dst.reshape(_NSC * _NTILE, nch_deg, _CH)
    zeros_nk = jnp.zeros((_N, 16), jnp.float32)
    ones_ck = jnp.ones((_CH, 16), jnp.float32)
    zeros_n = jnp.zeros((_N, 128), jnp.float32)

    degp = _sc_degree(dst_deg, zeros_nk, ones_ck)
    z1 = _tc_layer1(x, W1, degp)
    u1 = _sc_agg_feature(z1.reshape(2 * _N, 128), src_off, dst_agg)
    z2 = _tc_mid(u1.reshape(2, _N, 128), degp, b1, W2, split=True)
    u2 = _sc_agg_feature(z2.reshape(2 * _N, 128), src_off, dst_agg)
    z3 = _tc_mid(u2.reshape(2, _N, 128), degp, b2, W3, split=False)
    u3 = _sc_agg_edge(z3, zeros_n, src_e, dst_e)
    return _tc_final(u3.reshape(2, _N, 128), degp, b3)


# speedup vs baseline: 19.8919x; 1.0061x over previous
import functools

import jax
import jax.numpy as jnp
from jax import lax
from jax.experimental import pallas as pl
from jax.experimental.pallas import tpu as pltpu
from jax.experimental.pallas import tpu_sc as plsc

_N = 10000
_E = 320000
_D_IN = 128
_D_H = 256
_D_OUT = 128

_NSC = 2
_NTILE = 16
_CH = 125
_RPT = 624
_TAIL0 = _RPT * _NTILE
_TAILN = _N - _TAIL0
_RB = 1000
_NBLK = _N // _RB


def _sc_mesh():
    return plsc.VectorSubcoreMesh(core_axis_name="c", subcore_axis_name="s")


def _rowsplit_copy(s, fn):
    fn(pl.multiple_of(s * _RPT, _RPT), _RPT)

    @pl.when(s == _NTILE - 1)
    def _():
        fn(_TAIL0, _TAILN)


def _sc_degree(dst_t, zeros_nk, ones_ck):
    nch = dst_t.shape[1]

    @functools.partial(
        pl.kernel,
        out_type=jax.ShapeDtypeStruct((_NSC, _N, 16), jnp.float32),
        mesh=_sc_mesh(),
        compiler_params=pltpu.CompilerParams(use_tc_tiling_on_sc=False),
        scratch_types=[
            pltpu.VMEM((nch, _CH), jnp.int32),
            pltpu.VMEM((_CH, 16), jnp.float32),
            pltpu.VMEM_SHARED((_N, 16), jnp.float32),
            pltpu.SemaphoreType.DMA,
        ],
    )
    def k(dst_hbm, zeros_hbm, ones_hbm, out_hbm, idx_v, ones_v, acc, sem):
        c = lax.axis_index("c")
        s = lax.axis_index("s")
        wid = c * _NTILE + s
        _rowsplit_copy(s, lambda o, n: pltpu.sync_copy(
            zeros_hbm.at[pl.ds(o, n)], acc.at[pl.ds(o, n)]))
        pltpu.sync_copy(dst_hbm.at[wid], idx_v)
        pltpu.sync_copy(ones_hbm, ones_v)
        plsc.subcore_barrier()

        def body(j, carry):
            pltpu.sync_copy(ones_v, acc.at[idx_v.at[j]], add=True)
            return carry

        lax.fori_loop(0, nch, body, 0)
        plsc.subcore_barrier()

        @pl.when(c == 0)
        def _():
            _rowsplit_copy(s, lambda o, n: pltpu.sync_copy(
                acc.at[pl.ds(o, n)], out_hbm.at[0].at[pl.ds(o, n)]))

        @pl.when(c == 1)
        def _():
            _rowsplit_copy(s, lambda o, n: pltpu.sync_copy(
                acc.at[pl.ds(o, n)], out_hbm.at[1].at[pl.ds(o, n)]))

    return k(dst_t, zeros_nk, ones_ck)


def _edge_pipeline(table, src_slab, dst_slab, acc, sblk, dblk, gbuf,
                   isem, gsem, ssem, nch):
    nsb = nch // 16

    def idx_load(bi, p):
        off = pl.multiple_of(bi * 8, 8)
        pltpu.async_copy(src_slab.at[pl.ds(off, 8)], sblk[p], isem[p])
        pltpu.async_copy(dst_slab.at[pl.ds(off, 8)], dblk[p], isem[p])

    def idx_wait(bi, p):
        off = pl.multiple_of(bi * 8, 8)
        pltpu.make_async_copy(src_slab.at[pl.ds(off, 8)], sblk[p], isem[p]).wait()
        pltpu.make_async_copy(dst_slab.at[pl.ds(off, 8)], dblk[p], isem[p]).wait()

    def g_issue(p, r, gp):
        pltpu.async_copy(table.at[sblk[p].at[r]], gbuf[gp], gsem[gp])

    def g_wait(p, r, gp):
        pltpu.make_async_copy(table.at[sblk[p].at[r]], gbuf[gp], gsem[gp]).wait()

    def s_issue(p, r, gp):
        pltpu.async_copy(gbuf[gp], acc.at[dblk[p].at[r]], ssem[gp], add=True)

    def s_wait(p, r, gp):
        pltpu.make_async_copy(gbuf[gp], acc.at[dblk[p].at[r]], ssem[gp]).wait()

    idx_load(0, 0)
    idx_wait(0, 0)
    g_issue(0, 0, 0)
    idx_load(1, 1)
    yield

    def body(k, carry):
        for m in range(16):
            gp = m % 2
            p = m // 8
            r = m % 8
            g_wait(p, r, gp)
            s_issue(p, r, gp)
            if m == 0:
                @pl.when(k > 0)
                def _(k=k):
                    s_wait(1, 7, 1)
                    idx_load(2 * k + 1, 1)
            else:
                pm = m - 1
                s_wait(pm // 8, pm % 8, 1 - gp)
            if m == 7:
                idx_wait(2 * k + 1, 1)
            if m == 8:
                @pl.when(k < nsb - 1)
                def _(k=k):
                    idx_load(2 * k + 2, 0)
            if m < 15:
                nm = m + 1
                g_issue(nm // 8, nm % 8, nm % 2)
            else:
                @pl.when(k < nsb - 1)
                def _(k=k):
                    idx_wait(2 * k + 2, 0)
                    g_issue(0, 0, 0)
        return carry

    lax.fori_loop(0, nsb, body, 0)
    s_wait(1, 7, 1)


_AGG_SCRATCH = [
    [pltpu.VMEM((8, _CH), jnp.int32)] * 2,
    [pltpu.VMEM((8, _CH), jnp.int32)] * 2,
    [pltpu.VMEM((_CH, 128), jnp.float32)] * 2,
    pltpu.VMEM_SHARED((_N, 128), jnp.float32),
    [pltpu.SemaphoreType.DMA] * 2,
    [pltpu.SemaphoreType.DMA] * 2,
    [pltpu.SemaphoreType.DMA] * 2,
]


def _sc_agg_feature(zcat, src_off, dst_t):
    nch = dst_t.shape[1]

    @functools.partial(
        pl.kernel,
        out_type=jax.ShapeDtypeStruct((2 * _N, 128), jnp.float32),
        mesh=_sc_mesh(),
        scratch_types=_AGG_SCRATCH,
    )
    def k(z_hbm, src_hbm, dst_hbm, out_hbm, sblk, dblk, gbuf, acc,
          isem, gsem, ssem):
        c = lax.axis_index("c")
        s = lax.axis_index("s")
        wid = c * _NTILE + s
        base = pl.multiple_of(c * _N, 8)
        pipe = _edge_pipeline(z_hbm, src_hbm.at[wid], dst_hbm.at[s], acc,
                              sblk, dblk, gbuf, isem, gsem, ssem, nch)
        next(pipe)
        _rowsplit_copy(s, lambda o, n: pltpu.sync_copy(
            z_hbm.at[pl.ds(pl.multiple_of(base + o, 8), n)], acc.at[pl.ds(o, n)]))
        plsc.subcore_barrier()
        next(pipe, None)
        plsc.subcore_barrier()
        _rowsplit_copy(s, lambda o, n: pltpu.sync_copy(
            acc.at[pl.ds(o, n)], out_hbm.at[pl.ds(pl.multiple_of(base + o, 8), n)]))

    return k(zcat, src_off, dst_t)


def _sc_agg_edge(z3, zeros_n, src_t, dst_t):
    nch = dst_t.shape[1]

    @functools.partial(
        pl.kernel,
        out_type=jax.ShapeDtypeStruct((2 * _N, 128), jnp.float32),
        mesh=_sc_mesh(),
        scratch_types=_AGG_SCRATCH,
    )
    def k(z_hbm, zeros_hbm, src_hbm, dst_hbm, out_hbm, sblk, dblk, gbuf, acc,
          isem, gsem, ssem):
        c = lax.axis_index("c")
        s = lax.axis_index("s")
        wid = c * _NTILE + s
        base = pl.multiple_of(c * _N, 8)
        pipe = _edge_pipeline(z_hbm, src_hbm.at[wid], dst_hbm.at[wid], acc,
                              sblk, dblk, gbuf, isem, gsem, ssem, nch)
        next(pipe)

        @pl.when(c == 0)
        def _():
            _rowsplit_copy(s, lambda o, n: pltpu.sync_copy(
                z_hbm.at[pl.ds(o, n)], acc.at[pl.ds(o, n)]))

        @pl.when(c == 1)
        def _():
            _rowsplit_copy(s, lambda o, n: pltpu.sync_copy(
                zeros_hbm.at[pl.ds(o, n)], acc.at[pl.ds(o, n)]))

        plsc.subcore_barrier()
        next(pipe, None)
        plsc.subcore_barrier()
        _rowsplit_copy(s, lambda o, n: pltpu.sync_copy(
            acc.at[pl.ds(o, n)], out_hbm.at[pl.ds(pl.multiple_of(base + o, 8), n)]))

    return k(z3, zeros_n, src_t, dst_t)


def _dinv_from(deg_blk):
    deg = deg_blk[0, :, 0:1] + deg_blk[1, :, 0:1] + 1.0
    return lax.rsqrt(deg)


def _tc_layer1(x, w1, degp):
    def body(x_ref, deg_ref, w_ref, out_ref):
        dinv = _dinv_from(deg_ref[...])
        z = dinv * jnp.dot(x_ref[...], w_ref[...], preferred_element_type=jnp.float32)
        out_ref[0, :, :] = z[:, : _D_H // 2]
        out_ref[1, :, :] = z[:, _D_H // 2:]

    return pl.pallas_call(
        body,
        grid=(_NBLK,),
        in_specs=[
            pl.BlockSpec((_RB, _D_IN), lambda i: (i, 0)),
            pl.BlockSpec((_NSC, _RB, 16), lambda i: (0, i, 0)),
            pl.BlockSpec((_D_IN, _D_H), lambda i: (0, 0)),
        ],
        out_specs=pl.BlockSpec((2, _RB, _D_H // 2), lambda i: (0, i, 0)),
        out_shape=jax.ShapeDtypeStruct((2, _N, _D_H // 2), jnp.float32),
    )(x, degp, w1)


def _tc_mid(up, degp, b, w, split):
    d_in = up.shape[2] * 2
    d_out = w.shape[1]

    def body(u_ref, deg_ref, b_ref, w_ref, out_ref):
        dinv = _dinv_from(deg_ref[...])
        u = u_ref[...]
        uc = jnp.concatenate([u[0], u[1]], axis=1)
        h = jnp.maximum(dinv * uc + b_ref[...], 0.0)
        z = dinv * jnp.dot(h, w_ref[...], preferred_element_type=jnp.float32)
        if split:
            out_ref[0, :, :] = z[:, : d_out // 2]
            out_ref[1, :, :] = z[:, d_out // 2:]
        else:
            out_ref[...] = z

    if split:
        out_specs = pl.BlockSpec((2, _RB, d_out // 2), lambda i: (0, i, 0))
        out_shape = jax.ShapeDtypeStruct((2, _N, d_out // 2), jnp.float32)
    else:
        out_specs = pl.BlockSpec((_RB, d_out), lambda i: (i, 0))
        out_shape = jax.ShapeDtypeStruct((_N, d_out), jnp.float32)

    return pl.pallas_call(
        body,
        grid=(_NBLK,),
        in_specs=[
            pl.BlockSpec((2, _RB, d_in // 2), lambda i: (0, i, 0)),
            pl.BlockSpec((_NSC, _RB, 16), lambda i: (0, i, 0)),
            pl.BlockSpec((1, d_in), lambda i: (0, 0)),
            pl.BlockSpec((d_in, d_out), lambda i: (0, 0)),
        ],
        out_specs=out_specs,
        out_shape=out_shape,
    )(up, degp, b.reshape(1, -1), w)


def _tc_final(up, degp, b):
    d_out = up.shape[2]

    def body(u_ref, deg_ref, b_ref, out_ref):
        dinv = _dinv_from(deg_ref[...])
        u = u_ref[...]
        out_ref[...] = dinv * (u[0] + u[1]) + b_ref[...]

    return pl.pallas_call(
        body,
        grid=(_NBLK,),
        in_specs=[
            pl.BlockSpec((2, _RB, d_out), lambda i: (0, i, 0)),
            pl.BlockSpec((_NSC, _RB, 16), lambda i: (0, i, 0)),
            pl.BlockSpec((1, d_out), lambda i: (0, 0)),
        ],
        out_specs=pl.BlockSpec((_RB, d_out), lambda i: (i, 0)),
        out_shape=jax.ShapeDtypeStruct((_N, d_out), jnp.float32),
    )(up, degp, b.reshape(1, -1))


def kernel(x, edge_index, W1, b1, W2, b2, W3, b3):
    src = edge_index[0]
    dst = edge_index[1]
    nch_deg = _E // (_NSC * _NTILE * _CH)
    nch_agg = _E // (_NTILE * _CH)
    dst_deg = dst.reshape(_NSC * _NTILE, nch_deg, _CH)
    src_agg = src.reshape(_NTILE, nch_agg, _CH)
    dst_agg = dst.reshape(_NTILE, nch_agg, _CH)
    src_off = jnp.concatenate([src_agg, src_agg + _N], axis=0)
    src_e = src.reshape(_NSC * _NTILE, nch_deg, _CH)
    dst_e = dst.reshape(_NSC * _NTILE, nch_deg, _CH)
    zeros_nk = jnp.zeros((_N, 16), jnp.float32)
    ones_ck = jnp.ones((_CH, 16), jnp.float32)
    zeros_n = jnp.zeros((_N, 128), jnp.float32)

    degp = _sc_degree(dst_deg, zeros_nk, ones_ck)
    z1 = _tc_layer1(x, W1, degp)
    u1 = _sc_agg_feature(z1.reshape(2 * _N, 128), src_off, dst_agg)
    z2 = _tc_mid(u1.reshape(2, _N, 128), degp, b1, W2, split=True)
    u2 = _sc_agg_feature(z2.reshape(2 * _N, 128), src_off, dst_agg)
    z3 = _tc_mid(u2.reshape(2, _N, 128), degp, b2, W3, split=False)
    u3 = _sc_agg_edge(z3, zeros_n, src_e, dst_e)
    return _tc_final(u3.reshape(2, _N, 128), degp, b3)


# speedup vs baseline: 23.5766x; 1.1852x over previous
import functools

import jax
import jax.numpy as jnp
from jax import lax
from jax.experimental import pallas as pl
from jax.experimental.pallas import tpu as pltpu
from jax.experimental.pallas import tpu_sc as plsc

_N = 10000
_E = 320000
_D_IN = 128
_D_H = 256
_D_OUT = 128

_NSC = 2
_NTILE = 16
_CH = 125
_RPT = 624
_TAIL0 = _RPT * _NTILE
_TAILN = _N - _TAIL0
_RB = 1000
_NBLK = _N // _RB


def _sc_mesh():
    return plsc.VectorSubcoreMesh(core_axis_name="c", subcore_axis_name="s")


def _rowsplit_copy(s, fn):
    fn(pl.multiple_of(s * _RPT, _RPT), _RPT)

    @pl.when(s == _NTILE - 1)
    def _():
        fn(_TAIL0, _TAILN)


def _sc_degree(dst_t, zeros_nk, ones_ck):
    nch = dst_t.shape[1]

    @functools.partial(
        pl.kernel,
        out_type=jax.ShapeDtypeStruct((_NSC, _N, 16), jnp.float32),
        mesh=_sc_mesh(),
        compiler_params=pltpu.CompilerParams(use_tc_tiling_on_sc=False),
        scratch_types=[
            pltpu.VMEM((nch, _CH), jnp.int32),
            pltpu.VMEM((_CH, 16), jnp.float32),
            pltpu.VMEM_SHARED((_N, 16), jnp.float32),
            pltpu.SemaphoreType.DMA,
        ],
    )
    def k(dst_hbm, zeros_hbm, ones_hbm, out_hbm, idx_v, ones_v, acc, sem):
        c = lax.axis_index("c")
        s = lax.axis_index("s")
        wid = c * _NTILE + s
        _rowsplit_copy(s, lambda o, n: pltpu.sync_copy(
            zeros_hbm.at[pl.ds(o, n)], acc.at[pl.ds(o, n)]))
        pltpu.sync_copy(dst_hbm.at[wid], idx_v)
        pltpu.sync_copy(ones_hbm, ones_v)
        plsc.subcore_barrier()

        def body(j, carry):
            pltpu.sync_copy(ones_v, acc.at[idx_v.at[j]], add=True)
            return carry

        lax.fori_loop(0, nch, body, 0)
        plsc.subcore_barrier()

        @pl.when(c == 0)
        def _():
            _rowsplit_copy(s, lambda o, n: pltpu.sync_copy(
                acc.at[pl.ds(o, n)], out_hbm.at[0].at[pl.ds(o, n)]))

        @pl.when(c == 1)
        def _():
            _rowsplit_copy(s, lambda o, n: pltpu.sync_copy(
                acc.at[pl.ds(o, n)], out_hbm.at[1].at[pl.ds(o, n)]))

    return k(dst_t, zeros_nk, ones_ck)


def _edge_pipeline(table, src_slab, dst_slab, acc, sblk, dblk, gbuf,
                   isem, gsem, ssem, nch):
    nsb = nch // 16

    def idx_load(bi, p):
        off = pl.multiple_of(bi * 8, 8)
        pltpu.async_copy(src_slab.at[pl.ds(off, 8)], sblk[p], isem[p])
        pltpu.async_copy(dst_slab.at[pl.ds(off, 8)], dblk[p], isem[p])

    def idx_wait(bi, p):
        off = pl.multiple_of(bi * 8, 8)
        pltpu.make_async_copy(src_slab.at[pl.ds(off, 8)], sblk[p], isem[p]).wait()
        pltpu.make_async_copy(dst_slab.at[pl.ds(off, 8)], dblk[p], isem[p]).wait()

    def g_issue(p, r, gp):
        pltpu.async_copy(table.at[sblk[p].at[r]], gbuf[gp], gsem[gp])

    def g_wait(p, r, gp):
        pltpu.make_async_copy(table.at[sblk[p].at[r]], gbuf[gp], gsem[gp]).wait()

    def s_issue(p, r, gp):
        pltpu.async_copy(gbuf[gp], acc.at[dblk[p].at[r]], ssem[gp], add=True)

    def s_wait(p, r, gp):
        pltpu.make_async_copy(gbuf[gp], acc.at[dblk[p].at[r]], ssem[gp]).wait()

    idx_load(0, 0)
    idx_wait(0, 0)
    g_issue(0, 0, 0)
    idx_load(1, 1)
    yield

    def body(k, carry):
        for m in range(16):
            gp = m % 2
            p = m // 8
            r = m % 8
            if m == 0:
                @pl.when(k > 0)
                def _(k=k):
                    s_wait(1, 7, 1)
                    idx_load(2 * k + 1, 1)
            else:
                pm = m - 1
                s_wait(pm // 8, pm % 8, 1 - gp)
            if m == 7:
                idx_wait(2 * k + 1, 1)
            if m == 8:
                @pl.when(k < nsb - 1)
                def _(k=k):
                    idx_load(2 * k + 2, 0)
            if m < 15:
                nm = m + 1
                g_issue(nm // 8, nm % 8, nm % 2)
            else:
                @pl.when(k < nsb - 1)
                def _(k=k):
                    idx_wait(2 * k + 2, 0)
                    g_issue(0, 0, 0)
            g_wait(p, r, gp)
            s_issue(p, r, gp)
        return carry

    lax.fori_loop(0, nsb, body, 0)
    s_wait(1, 7, 1)


_AGG_SCRATCH = [
    [pltpu.VMEM((8, _CH), jnp.int32)] * 2,
    [pltpu.VMEM((8, _CH), jnp.int32)] * 2,
    [pltpu.VMEM((_CH, 128), jnp.float32)] * 2,
    pltpu.VMEM_SHARED((_N, 128), jnp.float32),
    [pltpu.SemaphoreType.DMA] * 2,
    [pltpu.SemaphoreType.DMA] * 2,
    [pltpu.SemaphoreType.DMA] * 2,
]


def _sc_agg_feature(zcat, src_off, dst_t):
    nch = dst_t.shape[1]

    @functools.partial(
        pl.kernel,
        out_type=jax.ShapeDtypeStruct((2 * _N, 128), jnp.float32),
        mesh=_sc_mesh(),
        scratch_types=_AGG_SCRATCH,
    )
    def k(z_hbm, src_hbm, dst_hbm, out_hbm, sblk, dblk, gbuf, acc,
          isem, gsem, ssem):
        c = lax.axis_index("c")
        s = lax.axis_index("s")
        wid = c * _NTILE + s
        base = pl.multiple_of(c * _N, 8)
        pipe = _edge_pipeline(z_hbm, src_hbm.at[wid], dst_hbm.at[s], acc,
                              sblk, dblk, gbuf, isem, gsem, ssem, nch)
        next(pipe)
        _rowsplit_copy(s, lambda o, n: pltpu.sync_copy(
            z_hbm.at[pl.ds(pl.multiple_of(base + o, 8), n)], acc.at[pl.ds(o, n)]))
        plsc.subcore_barrier()
        next(pipe, None)
        plsc.subcore_barrier()
        _rowsplit_copy(s, lambda o, n: pltpu.sync_copy(
            acc.at[pl.ds(o, n)], out_hbm.at[pl.ds(pl.multiple_of(base + o, 8), n)]))

    return k(zcat, src_off, dst_t)


def _sc_agg_edge(z3, zeros_n, src_t, dst_t):
    nch = dst_t.shape[1]

    @functools.partial(
        pl.kernel,
        out_type=jax.ShapeDtypeStruct((2 * _N, 128), jnp.float32),
        mesh=_sc_mesh(),
        scratch_types=_AGG_SCRATCH,
    )
    def k(z_hbm, zeros_hbm, src_hbm, dst_hbm, out_hbm, sblk, dblk, gbuf, acc,
          isem, gsem, ssem):
        c = lax.axis_index("c")
        s = lax.axis_index("s")
        wid = c * _NTILE + s
        base = pl.multiple_of(c * _N, 8)
        pipe = _edge_pipeline(z_hbm, src_hbm.at[wid], dst_hbm.at[wid], acc,
                              sblk, dblk, gbuf, isem, gsem, ssem, nch)
        next(pipe)

        @pl.when(c == 0)
        def _():
            _rowsplit_copy(s, lambda o, n: pltpu.sync_copy(
                z_hbm.at[pl.ds(o, n)], acc.at[pl.ds(o, n)]))

        @pl.when(c == 1)
        def _():
            _rowsplit_copy(s, lambda o, n: pltpu.sync_copy(
                zeros_hbm.at[pl.ds(o, n)], acc.at[pl.ds(o, n)]))

        plsc.subcore_barrier()
        next(pipe, None)
        plsc.subcore_barrier()
        _rowsplit_copy(s, lambda o, n: pltpu.sync_copy(
            acc.at[pl.ds(o, n)], out_hbm.at[pl.ds(pl.multiple_of(base + o, 8), n)]))

    return k(z3, zeros_n, src_t, dst_t)


def _dinv_from(deg_blk):
    deg = deg_blk[0, :, 0:1] + deg_blk[1, :, 0:1] + 1.0
    return lax.rsqrt(deg)


def _tc_layer1(x, w1, degp):
    def body(x_ref, deg_ref, w_ref, out_ref):
        dinv = _dinv_from(deg_ref[...])
        z = dinv * jnp.dot(x_ref[...], w_ref[...], preferred_element_type=jnp.float32)
        out_ref[0, :, :] = z[:, : _D_H // 2]
        out_ref[1, :, :] = z[:, _D_H // 2:]

    return pl.pallas_call(
        body,
        grid=(_NBLK,),
        in_specs=[
            pl.BlockSpec((_RB, _D_IN), lambda i: (i, 0)),
            pl.BlockSpec((_NSC, _RB, 16), lambda i: (0, i, 0)),
            pl.BlockSpec((_D_IN, _D_H), lambda i: (0, 0)),
        ],
        out_specs=pl.BlockSpec((2, _RB, _D_H // 2), lambda i: (0, i, 0)),
        out_shape=jax.ShapeDtypeStruct((2, _N, _D_H // 2), jnp.float32),
    )(x, degp, w1)


def _tc_mid(up, degp, b, w, split):
    d_in = up.shape[2] * 2
    d_out = w.shape[1]

    def body(u_ref, deg_ref, b_ref, w_ref, out_ref):
        dinv = _dinv_from(deg_ref[...])
        u = u_ref[...]
        uc = jnp.concatenate([u[0], u[1]], axis=1)
        h = jnp.maximum(dinv * uc + b_ref[...], 0.0)
        z = dinv * jnp.dot(h, w_ref[...], preferred_element_type=jnp.float32)
        if split:
            out_ref[0, :, :] = z[:, : d_out // 2]
            out_ref[1, :, :] = z[:, d_out // 2:]
        else:
            out_ref[...] = z

    if split:
        out_specs = pl.BlockSpec((2, _RB, d_out // 2), lambda i: (0, i, 0))
        out_shape = jax.ShapeDtypeStruct((2, _N, d_out // 2), jnp.float32)
    else:
        out_specs = pl.BlockSpec((_RB, d_out), lambda i: (i, 0))
        out_shape = jax.ShapeDtypeStruct((_N, d_out), jnp.float32)

    return pl.pallas_call(
        body,
        grid=(_NBLK,),
        in_specs=[
            pl.BlockSpec((2, _RB, d_in // 2), lambda i: (0, i, 0)),
            pl.BlockSpec((_NSC, _RB, 16), lambda i: (0, i, 0)),
            pl.BlockSpec((1, d_in), lambda i: (0, 0)),
            pl.BlockSpec((d_in, d_out), lambda i: (0, 0)),
        ],
        out_specs=out_specs,
        out_shape=out_shape,
    )(up, degp, b.reshape(1, -1), w)


def _tc_final(up, degp, b):
    d_out = up.shape[2]

    def body(u_ref, deg_ref, b_ref, out_ref):
        dinv = _dinv_from(deg_ref[...])
        u = u_ref[...]
        out_ref[...] = dinv * (u[0] + u[1]) + b_ref[...]

    return pl.pallas_call(
        body,
        grid=(_NBLK,),
        in_specs=[
            pl.BlockSpec((2, _RB, d_out), lambda i: (0, i, 0)),
            pl.BlockSpec((_NSC, _RB, 16), lambda i: (0, i, 0)),
            pl.BlockSpec((1, d_out), lambda i: (0, 0)),
        ],
        out_specs=pl.BlockSpec((_RB, d_out), lambda i: (i, 0)),
        out_shape=jax.ShapeDtypeStruct((_N, d_out), jnp.float32),
    )(up, degp, b.reshape(1, -1))


def kernel(x, edge_index, W1, b1, W2, b2, W3, b3):
    src = edge_index[0]
    dst = edge_index[1]
    nch_deg = _E // (_NSC * _NTILE * _CH)
    nch_agg = _E // (_NTILE * _CH)
    dst_deg = dst.reshape(_NSC * _NTILE, nch_deg, _CH)
    src_agg = src.reshape(_NTILE, nch_agg, _CH)
    dst_agg = dst.reshape(_NTILE, nch_agg, _CH)
    src_off = jnp.concatenate([src_agg, src_agg + _N], axis=0)
    src_e = src.reshape(_NSC * _NTILE, nch_deg, _CH)
    dst_e = dst.reshape(_NSC * _NTILE, nch_deg, _CH)
    zeros_nk = jnp.zeros((_N, 16), jnp.float32)
    ones_ck = jnp.ones((_CH, 16), jnp.float32)
    zeros_n = jnp.zeros((_N, 128), jnp.float32)

    degp = _sc_degree(dst_deg, zeros_nk, ones_ck)
    z1 = _tc_layer1(x, W1, degp)
    u1 = _sc_agg_feature(z1.reshape(2 * _N, 128), src_off, dst_agg)
    z2 = _tc_mid(u1.reshape(2, _N, 128), degp, b1, W2, split=True)
    u2 = _sc_agg_feature(z2.reshape(2 * _N, 128), src_off, dst_agg)
    z3 = _tc_mid(u2.reshape(2, _N, 128), degp, b2, W3, split=False)
    u3 = _sc_agg_edge(z3, zeros_n, src_e, dst_e)
    return _tc_final(u3.reshape(2, _N, 128), degp, b3)
